# Initial kernel scaffold; baseline (speedup 1.0000x reference)
import jax
import jax.numpy as jnp
from jax.experimental import pallas as pl


def kernel(x, edge_index, W1, b1, W2, b2):
    raise NotImplementedError("write your pallas kernel here")


# speedup vs baseline: 21.4013x; 21.4013x over previous
import functools

import jax
import jax.numpy as jnp
from jax import lax
from jax.experimental import pallas as pl
from jax.experimental.pallas import tpu as pltpu
from jax.experimental.pallas import tpu_sc as plsc

N = 10000
E = 320000
F_IN = 128
HID = 128
C = 7
CP = 16
CD = 8

NC = 2
NS = 16
NW = NC * NS
EPW = E // NW
CHUNK = 80
NCHUNK = EPW // CHUNK
NP = 10240
RPT = NP // NS

_MESH = plsc.VectorSubcoreMesh(core_axis_name="c", subcore_axis_name="s")
_SC_PARAMS = pltpu.CompilerParams(use_tc_tiling_on_sc=False)


def _edge_loop(g_hbm, src_v, dst_v, rows_v, acc_sh, gsem):
  first = pltpu.async_copy(g_hbm.at[src_v.at[0]], rows_v.at[0], gsem)
  first.wait()

  def body(j, _):
    slot = lax.rem(j, 2)
    nxt = pltpu.async_copy(g_hbm.at[src_v.at[j + 1]],
                           rows_v.at[1 - slot], gsem)
    pltpu.sync_copy(rows_v.at[slot], acc_sh.at[dst_v.at[j]], add=True)
    nxt.wait()
    return 0

  lax.fori_loop(0, NCHUNK - 1, body, 0)
  last = lax.rem(NCHUNK - 1, 2)
  pltpu.sync_copy(rows_v.at[last], acc_sh.at[dst_v.at[NCHUNK - 1]],
                  add=True)


HHID = HID // 2


@functools.partial(
    pl.kernel,
    out_type=jax.ShapeDtypeStruct((NC, 2, NP, HHID), jnp.float32),
    mesh=_MESH,
    compiler_params=_SC_PARAMS,
    scratch_types=[
        pltpu.VMEM((NCHUNK, CHUNK), jnp.int32),
        pltpu.VMEM((NCHUNK, CHUNK), jnp.int32),
        pltpu.VMEM((2, CHUNK, HHID), jnp.float32),
        pltpu.VMEM_SHARED((NP, HHID), jnp.float32),
        pltpu.SemaphoreType.DMA,
    ],
)
def _edge_pass_wide(ga_hbm, gb_hbm, src_hbm, dst_hbm, zeros_hbm, out_hbm,
                    src_v, dst_v, rows_v, acc_sh, gsem):
  cid = lax.axis_index("c")
  sid = lax.axis_index("s")
  wid = cid * NS + sid
  stripe = pl.ds(sid * RPT, RPT)
  pltpu.sync_copy(src_hbm.at[wid], src_v)
  pltpu.sync_copy(dst_hbm.at[wid], dst_v)
  pltpu.sync_copy(zeros_hbm.at[stripe], acc_sh.at[stripe])
  plsc.subcore_barrier()
  for h, g_hbm in enumerate((ga_hbm, gb_hbm)):
    _edge_loop(g_hbm, src_v, dst_v, rows_v, acc_sh, gsem)
    plsc.subcore_barrier()
    pltpu.sync_copy(acc_sh.at[stripe], out_hbm.at[cid, h, stripe])
    if h == 0:
      pltpu.sync_copy(zeros_hbm.at[stripe], acc_sh.at[stripe])
      plsc.subcore_barrier()


@functools.partial(
    pl.kernel,
    out_type=jax.ShapeDtypeStruct((NC, NP, CP), jnp.float32),
    mesh=_MESH,
    compiler_params=_SC_PARAMS,
    scratch_types=[
        pltpu.VMEM((NCHUNK, CHUNK), jnp.int32),
        pltpu.VMEM((NCHUNK, CHUNK), jnp.int32),
        pltpu.VMEM((2, CHUNK, CP), jnp.float32),
        pltpu.VMEM_SHARED((NP, CP), jnp.float32),
        pltpu.SemaphoreType.DMA,
    ],
)
def _edge_pass_narrow(g_hbm, src_hbm, dst_hbm, zeros_hbm, out_hbm,
                      src_v, dst_v, rows_v, acc_sh, gsem):
  cid = lax.axis_index("c")
  sid = lax.axis_index("s")
  wid = cid * NS + sid
  stripe = pl.ds(sid * RPT, RPT)
  pltpu.sync_copy(src_hbm.at[wid], src_v)
  pltpu.sync_copy(dst_hbm.at[wid], dst_v)
  pltpu.sync_copy(zeros_hbm.at[stripe], acc_sh.at[stripe])
  plsc.subcore_barrier()
  _edge_loop(g_hbm, src_v, dst_v, rows_v, acc_sh, gsem)
  plsc.subcore_barrier()
  pltpu.sync_copy(acc_sh.at[stripe], out_hbm.at[cid, stripe])


@functools.partial(
    pl.kernel,
    out_type=jax.ShapeDtypeStruct((NC, NP, CD), jnp.float32),
    mesh=_MESH,
    compiler_params=_SC_PARAMS,
    scratch_types=[
        pltpu.VMEM((NCHUNK, CHUNK), jnp.int32),
        pltpu.VMEM((CHUNK, CD), jnp.float32),
        pltpu.VMEM_SHARED((NP, CD), jnp.float32),
    ],
)
def _deg_pass(dst_hbm, ones_hbm, zeros_hbm, out_hbm,
              dst_v, ones_v, acc_sh):
  cid = lax.axis_index("c")
  sid = lax.axis_index("s")
  wid = cid * NS + sid
  pltpu.sync_copy(zeros_hbm.at[pl.ds(sid * RPT, RPT)],
                  acc_sh.at[pl.ds(sid * RPT, RPT)])
  pltpu.sync_copy(dst_hbm.at[wid], dst_v)
  pltpu.sync_copy(ones_hbm, ones_v)
  plsc.subcore_barrier()

  def body(j, _):
    pltpu.sync_copy(ones_v, acc_sh.at[dst_v.at[j]], add=True)
    return 0

  lax.fori_loop(0, NCHUNK, body, 0)
  plsc.subcore_barrier()
  pltpu.sync_copy(acc_sh.at[pl.ds(sid * RPT, RPT)],
                  out_hbm.at[cid, pl.ds(sid * RPT, RPT)])


BR = 1000


def _tc1_body(x_ref, w1_ref, degp_ref, ga_ref, gb_ref):
  deg = 1.0 + degp_ref[0, :, 0:1] + degp_ref[1, :, 0:1]
  dis = lax.rsqrt(deg)
  h = jnp.dot(x_ref[...], w1_ref[...], preferred_element_type=jnp.float32)
  g1 = h * dis
  ga_ref[...] = g1[:, :HHID]
  gb_ref[...] = g1[:, HHID:]


def _tc2_body(pp_ref, ga_ref, gb_ref, degp_ref, b1_ref, w2_ref, g2_ref):
  deg = 1.0 + degp_ref[0, :, 0:1] + degp_ref[1, :, 0:1]
  dis = lax.rsqrt(deg)
  g1 = jnp.concatenate([ga_ref[...], gb_ref[...]], axis=1)
  psum = jnp.concatenate(
      [pp_ref[0, 0] + pp_ref[1, 0], pp_ref[0, 1] + pp_ref[1, 1]], axis=1)
  agg = dis * (psum + g1) + b1_ref[...]
  h1 = jnp.maximum(agg, 0.0)
  h2 = jnp.dot(h1, w2_ref[...], preferred_element_type=jnp.float32)
  g2_ref[...] = h2 * dis


def _tc3_body(qq_ref, g2_ref, degp_ref, b2_ref, out_ref):
  deg = 1.0 + degp_ref[0, :, 0:1] + degp_ref[1, :, 0:1]
  dis = lax.rsqrt(deg)
  out_ref[...] = dis * (qq_ref[0] + qq_ref[1] + g2_ref[...]) + b2_ref[...]


def _row_blocked(body, out_w, in_specs):
  return pl.pallas_call(
      body,
      grid=(N // BR,),
      in_specs=in_specs,
      out_specs=pl.BlockSpec((BR, out_w), lambda i: (i, 0)),
      out_shape=jax.ShapeDtypeStruct((N, out_w), jnp.float32),
  )


def _spec_rows(w):
  return pl.BlockSpec((BR, w), lambda i: (i, 0))


def _spec_pair(w):
  return pl.BlockSpec((NC, BR, w), lambda i: (0, i, 0))


def _spec_full(r, w):
  return pl.BlockSpec((r, w), lambda i: (0, 0))


def kernel(x, edge_index, W1, b1, W2, b2):
  src = edge_index[0].reshape(NW, NCHUNK, CHUNK)
  dst = edge_index[1].reshape(NW, NCHUNK, CHUNK)
  zeros64 = jnp.zeros((NP, HID // 2), jnp.float32)
  zeros16 = jnp.zeros((NP, CP), jnp.float32)
  zeros8 = jnp.zeros((NP, CD), jnp.float32)
  ones = jnp.ones((CHUNK, CD), jnp.float32)
  w2p = jnp.pad(W2, ((0, 0), (0, CP - C)))
  b1r = b1.reshape(1, HID)
  b2r = jnp.pad(b2, (0, CP - C)).reshape(1, CP)

  degp = _deg_pass(dst, ones, zeros8)

  ga, gb = pl.pallas_call(
      _tc1_body,
      grid=(N // BR,),
      in_specs=[_spec_rows(F_IN), _spec_full(F_IN, HID), _spec_pair(CD)],
      out_specs=[_spec_rows(HHID), _spec_rows(HHID)],
      out_shape=[jax.ShapeDtypeStruct((N, HHID), jnp.float32)] * 2,
  )(x, W1, degp)

  pp1 = _edge_pass_wide(ga, gb, src, dst, zeros64)

  g2 = _row_blocked(
      _tc2_body, CP,
      [pl.BlockSpec((NC, 2, BR, HHID), lambda i: (0, 0, i, 0)),
       _spec_rows(HHID), _spec_rows(HHID), _spec_pair(CD),
       _spec_full(1, HID), _spec_full(HID, CP)],
  )(pp1, ga, gb, degp, b1r, w2p)

  pp2 = _edge_pass_narrow(g2, src, dst, zeros16)

  out16 = _row_blocked(
      _tc3_body, CP,
      [_spec_pair(CP), _spec_rows(CP), _spec_pair(CD), _spec_full(1, CP)],
  )(pp2, g2, degp, b2r)

  return out16[:, :C]


# speedup vs baseline: 33.0200x; 1.5429x over previous
import functools

import jax
import jax.numpy as jnp
from jax import lax
from jax.experimental import pallas as pl
from jax.experimental.pallas import tpu as pltpu
from jax.experimental.pallas import tpu_sc as plsc

N = 10000
E = 320000
F_IN = 128
HID = 128
C = 7
CP = 16
CD = 8

NC = 2
NS = 16
NW = NC * NS
EPW = E // NW
CHUNK = 80
NCHUNK = EPW // CHUNK
NP = 10240
RPT = NP // NS

_MESH = plsc.VectorSubcoreMesh(core_axis_name="c", subcore_axis_name="s")
_SC_PARAMS = pltpu.CompilerParams(use_tc_tiling_on_sc=False)


NBUF = 4
GRP = 25


def _edge_loop(g_hbm, src_v, dst_v, rows_v, acc_sh, gsem, ssem):
  def group(i, _):
    base = i * GRP
    gd, sd = {}, {}
    for k in range(2):
      gd[k] = pltpu.async_copy(g_hbm.at[src_v.at[base + k]],
                               rows_v.at[k], gsem)
    for k in range(GRP):
      if k >= 2:
        sd[k - 2].wait()
      if k + 2 < GRP:
        gd[k + 2] = pltpu.async_copy(g_hbm.at[src_v.at[base + k + 2]],
                                     rows_v.at[(k + 2) % NBUF], gsem)
      gd[k].wait()
      sd[k] = pltpu.async_copy(rows_v.at[k % NBUF],
                               acc_sh.at[dst_v.at[base + k]], ssem, add=True)
    sd[GRP - 2].wait()
    sd[GRP - 1].wait()
    return 0

  lax.fori_loop(0, NCHUNK // GRP, group, 0)


HHID = HID // 2


@functools.partial(
    pl.kernel,
    out_type=jax.ShapeDtypeStruct((NC, 2, NP, HHID), jnp.float32),
    mesh=_MESH,
    compiler_params=_SC_PARAMS,
    scratch_types=[
        pltpu.VMEM((NCHUNK, CHUNK), jnp.int32),
        pltpu.VMEM((NCHUNK, CHUNK), jnp.int32),
        pltpu.VMEM((NBUF, CHUNK, HHID), jnp.float32),
        pltpu.VMEM_SHARED((NP, HHID), jnp.float32),
        pltpu.SemaphoreType.DMA,
        pltpu.SemaphoreType.DMA,
    ],
)
def _edge_pass_wide(ga_hbm, gb_hbm, src_hbm, dst_hbm, zeros_hbm, out_hbm,
                    src_v, dst_v, rows_v, acc_sh, gsem, ssem):
  cid = lax.axis_index("c")
  sid = lax.axis_index("s")
  wid = cid * NS + sid
  stripe = pl.ds(sid * RPT, RPT)
  pltpu.sync_copy(src_hbm.at[wid], src_v)
  pltpu.sync_copy(dst_hbm.at[wid], dst_v)
  pltpu.sync_copy(zeros_hbm.at[stripe], acc_sh.at[stripe])
  plsc.subcore_barrier()
  for h, g_hbm in enumerate((ga_hbm, gb_hbm)):
    _edge_loop(g_hbm, src_v, dst_v, rows_v, acc_sh, gsem, ssem)
    plsc.subcore_barrier()
    pltpu.sync_copy(acc_sh.at[stripe], out_hbm.at[cid, h, stripe])
    if h == 0:
      pltpu.sync_copy(zeros_hbm.at[stripe], acc_sh.at[stripe])
      plsc.subcore_barrier()


@functools.partial(
    pl.kernel,
    out_type=jax.ShapeDtypeStruct((NC, NP, CP), jnp.float32),
    mesh=_MESH,
    compiler_params=_SC_PARAMS,
    scratch_types=[
        pltpu.VMEM((NCHUNK, CHUNK), jnp.int32),
        pltpu.VMEM((NCHUNK, CHUNK), jnp.int32),
        pltpu.VMEM((NBUF, CHUNK, CP), jnp.float32),
        pltpu.VMEM_SHARED((NP, CP), jnp.float32),
        pltpu.SemaphoreType.DMA,
        pltpu.SemaphoreType.DMA,
    ],
)
def _edge_pass_narrow(g_hbm, src_hbm, dst_hbm, zeros_hbm, out_hbm,
                      src_v, dst_v, rows_v, acc_sh, gsem, ssem):
  cid = lax.axis_index("c")
  sid = lax.axis_index("s")
  wid = cid * NS + sid
  stripe = pl.ds(sid * RPT, RPT)
  pltpu.sync_copy(src_hbm.at[wid], src_v)
  pltpu.sync_copy(dst_hbm.at[wid], dst_v)
  pltpu.sync_copy(zeros_hbm.at[stripe], acc_sh.at[stripe])
  plsc.subcore_barrier()
  _edge_loop(g_hbm, src_v, dst_v, rows_v, acc_sh, gsem, ssem)
  plsc.subcore_barrier()
  pltpu.sync_copy(acc_sh.at[stripe], out_hbm.at[cid, stripe])


@functools.partial(
    pl.kernel,
    out_type=jax.ShapeDtypeStruct((NC, NP, CD), jnp.float32),
    mesh=_MESH,
    compiler_params=_SC_PARAMS,
    scratch_types=[
        pltpu.VMEM((NCHUNK, CHUNK), jnp.int32),
        pltpu.VMEM((CHUNK, CD), jnp.float32),
        pltpu.VMEM_SHARED((NP, CD), jnp.float32),
        pltpu.SemaphoreType.DMA,
    ],
)
def _deg_pass(dst_hbm, ones_hbm, zeros_hbm, out_hbm,
              dst_v, ones_v, acc_sh, dsem):
  cid = lax.axis_index("c")
  sid = lax.axis_index("s")
  wid = cid * NS + sid
  pltpu.sync_copy(zeros_hbm.at[pl.ds(sid * RPT, RPT)],
                  acc_sh.at[pl.ds(sid * RPT, RPT)])
  pltpu.sync_copy(dst_hbm.at[wid], dst_v)
  pltpu.sync_copy(ones_hbm, ones_v)
  plsc.subcore_barrier()

  def group(i, _):
    sd = {}
    for k in range(GRP):
      if k >= 3:
        sd[k - 3].wait()
      sd[k] = pltpu.async_copy(ones_v, acc_sh.at[dst_v.at[i * GRP + k]],
                               dsem, add=True)
    for k in range(GRP - 3, GRP):
      sd[k].wait()
    return 0

  lax.fori_loop(0, NCHUNK // GRP, group, 0)
  plsc.subcore_barrier()
  pltpu.sync_copy(acc_sh.at[pl.ds(sid * RPT, RPT)],
                  out_hbm.at[cid, pl.ds(sid * RPT, RPT)])


BR = 1000


def _tc1_body(x_ref, w1_ref, degp_ref, ga_ref, gb_ref):
  deg = 1.0 + degp_ref[0, :, 0:1] + degp_ref[1, :, 0:1]
  dis = lax.rsqrt(deg)
  h = jnp.dot(x_ref[...], w1_ref[...], preferred_element_type=jnp.float32)
  g1 = h * dis
  ga_ref[...] = g1[:, :HHID]
  gb_ref[...] = g1[:, HHID:]


def _tc2_body(pp_ref, ga_ref, gb_ref, degp_ref, b1_ref, w2_ref, g2_ref):
  deg = 1.0 + degp_ref[0, :, 0:1] + degp_ref[1, :, 0:1]
  dis = lax.rsqrt(deg)
  g1 = jnp.concatenate([ga_ref[...], gb_ref[...]], axis=1)
  psum = jnp.concatenate(
      [pp_ref[0, 0] + pp_ref[1, 0], pp_ref[0, 1] + pp_ref[1, 1]], axis=1)
  agg = dis * (psum + g1) + b1_ref[...]
  h1 = jnp.maximum(agg, 0.0)
  h2 = jnp.dot(h1, w2_ref[...], preferred_element_type=jnp.float32)
  g2_ref[...] = h2 * dis


def _tc3_body(qq_ref, g2_ref, degp_ref, b2_ref, out_ref):
  deg = 1.0 + degp_ref[0, :, 0:1] + degp_ref[1, :, 0:1]
  dis = lax.rsqrt(deg)
  out_ref[...] = dis * (qq_ref[0] + qq_ref[1] + g2_ref[...]) + b2_ref[...]


def _row_blocked(body, out_w, in_specs):
  return pl.pallas_call(
      body,
      grid=(N // BR,),
      in_specs=in_specs,
      out_specs=pl.BlockSpec((BR, out_w), lambda i: (i, 0)),
      out_shape=jax.ShapeDtypeStruct((N, out_w), jnp.float32),
  )


def _spec_rows(w):
  return pl.BlockSpec((BR, w), lambda i: (i, 0))


def _spec_pair(w):
  return pl.BlockSpec((NC, BR, w), lambda i: (0, i, 0))


def _spec_full(r, w):
  return pl.BlockSpec((r, w), lambda i: (0, 0))


def kernel(x, edge_index, W1, b1, W2, b2):
  src = edge_index[0].reshape(NW, NCHUNK, CHUNK)
  dst = edge_index[1].reshape(NW, NCHUNK, CHUNK)
  zeros64 = jnp.zeros((NP, HID // 2), jnp.float32)
  zeros16 = jnp.zeros((NP, CP), jnp.float32)
  zeros8 = jnp.zeros((NP, CD), jnp.float32)
  ones = jnp.ones((CHUNK, CD), jnp.float32)
  w2p = jnp.pad(W2, ((0, 0), (0, CP - C)))
  b1r = b1.reshape(1, HID)
  b2r = jnp.pad(b2, (0, CP - C)).reshape(1, CP)

  degp = _deg_pass(dst, ones, zeros8)

  ga, gb = pl.pallas_call(
      _tc1_body,
      grid=(N // BR,),
      in_specs=[_spec_rows(F_IN), _spec_full(F_IN, HID), _spec_pair(CD)],
      out_specs=[_spec_rows(HHID), _spec_rows(HHID)],
      out_shape=[jax.ShapeDtypeStruct((N, HHID), jnp.float32)] * 2,
  )(x, W1, degp)

  pp1 = _edge_pass_wide(ga, gb, src, dst, zeros64)

  g2 = _row_blocked(
      _tc2_body, CP,
      [pl.BlockSpec((NC, 2, BR, HHID), lambda i: (0, 0, i, 0)),
       _spec_rows(HHID), _spec_rows(HHID), _spec_pair(CD),
       _spec_full(1, HID), _spec_full(HID, CP)],
  )(pp1, ga, gb, degp, b1r, w2p)

  pp2 = _edge_pass_narrow(g2, src, dst, zeros16)

  out16 = _row_blocked(
      _tc3_body, CP,
      [_spec_pair(CP), _spec_rows(CP), _spec_pair(CD), _spec_full(1, CP)],
  )(pp2, g2, degp, b2r)

  return out16[:, :C]


# speedup vs baseline: 37.1388x; 1.1247x over previous
import functools

import jax
import jax.numpy as jnp
from jax import lax
from jax.experimental import pallas as pl
from jax.experimental.pallas import tpu as pltpu
from jax.experimental.pallas import tpu_sc as plsc

N = 10000
E = 320000
F_IN = 128
HID = 128
C = 7
CP = 16
PK = 8

NC = 2
NS = 16
NW = NC * NS
EPW = E // NW
CHUNK = 80
NCHUNK = EPW // CHUNK
NP = 10240
RPT = NP // NS
NPQ = NP // PK
HH = HID // 2

_MESH = plsc.VectorSubcoreMesh(core_axis_name="c", subcore_axis_name="s")
_SC_PARAMS = pltpu.CompilerParams(use_tc_tiling_on_sc=False)

NBUF = 4
GRP = 25


def _edge_loop(g_hbm, src_v, dst_v, rows_v, acc_sh, gsem, ssem):
  def group(i, _):
    base = i * GRP
    gd, sd = {}, {}
    for k in range(2):
      gd[k] = pltpu.async_copy(g_hbm.at[src_v.at[base + k]],
                               rows_v.at[k], gsem)
    for k in range(GRP):
      if k >= 2:
        sd[k - 2].wait()
      if k + 2 < GRP:
        gd[k + 2] = pltpu.async_copy(g_hbm.at[src_v.at[base + k + 2]],
                                     rows_v.at[(k + 2) % NBUF], gsem)
      gd[k].wait()
      sd[k] = pltpu.async_copy(rows_v.at[k % NBUF],
                               acc_sh.at[dst_v.at[base + k]], ssem, add=True)
    sd[GRP - 2].wait()
    sd[GRP - 1].wait()
    return 0

  lax.fori_loop(0, NCHUNK // GRP, group, 0)


@functools.partial(
    pl.kernel,
    out_type=jax.ShapeDtypeStruct((NC, 2, NP, HH), jnp.float32),
    mesh=_MESH,
    compiler_params=_SC_PARAMS,
    scratch_types=[
        pltpu.VMEM((NCHUNK, CHUNK), jnp.int32),
        pltpu.VMEM((NCHUNK, CHUNK), jnp.int32),
        pltpu.VMEM((NCHUNK, CHUNK), jnp.int32),
        pltpu.VMEM((NBUF, CHUNK, HH), jnp.float32),
        pltpu.VMEM_SHARED((NP, HH), jnp.float32),
        pltpu.SemaphoreType.DMA,
        pltpu.SemaphoreType.DMA,
    ],
)
def _edge_pass_wide(g2n_hbm, srca_hbm, srcb_hbm, dst_hbm, zeros_hbm, out_hbm,
                    srca_v, srcb_v, dst_v, rows_v, acc_sh, gsem, ssem):
  cid = lax.axis_index("c")
  sid = lax.axis_index("s")
  wid = cid * NS + sid
  stripe = pl.ds(sid * RPT, RPT)
  pltpu.sync_copy(srca_hbm.at[wid], srca_v)
  pltpu.sync_copy(srcb_hbm.at[wid], srcb_v)
  pltpu.sync_copy(dst_hbm.at[wid], dst_v)
  pltpu.sync_copy(zeros_hbm.at[stripe], acc_sh.at[stripe])
  plsc.subcore_barrier()
  for h, src_v in enumerate((srca_v, srcb_v)):
    _edge_loop(g2n_hbm, src_v, dst_v, rows_v, acc_sh, gsem, ssem)
    plsc.subcore_barrier()
    pltpu.sync_copy(acc_sh.at[stripe], out_hbm.at[cid, h, stripe])
    if h == 0:
      pltpu.sync_copy(zeros_hbm.at[stripe], acc_sh.at[stripe])
      plsc.subcore_barrier()


@functools.partial(
    pl.kernel,
    out_type=jax.ShapeDtypeStruct((NC, NP, CP), jnp.float32),
    mesh=_MESH,
    compiler_params=_SC_PARAMS,
    scratch_types=[
        pltpu.VMEM((NCHUNK, CHUNK), jnp.int32),
        pltpu.VMEM((NCHUNK, CHUNK), jnp.int32),
        pltpu.VMEM((NBUF, CHUNK, CP), jnp.float32),
        pltpu.VMEM_SHARED((NP, CP), jnp.float32),
        pltpu.SemaphoreType.DMA,
        pltpu.SemaphoreType.DMA,
    ],
)
def _edge_pass_narrow(g_hbm, src_hbm, dst_hbm, zeros_hbm, out_hbm,
                      src_v, dst_v, rows_v, acc_sh, gsem, ssem):
  cid = lax.axis_index("c")
  sid = lax.axis_index("s")
  wid = cid * NS + sid
  stripe = pl.ds(sid * RPT, RPT)
  pltpu.sync_copy(src_hbm.at[wid], src_v)
  pltpu.sync_copy(dst_hbm.at[wid], dst_v)
  pltpu.sync_copy(zeros_hbm.at[stripe], acc_sh.at[stripe])
  plsc.subcore_barrier()
  _edge_loop(g_hbm, src_v, dst_v, rows_v, acc_sh, gsem, ssem)
  plsc.subcore_barrier()
  pltpu.sync_copy(acc_sh.at[stripe], out_hbm.at[cid, stripe])


@functools.partial(
    pl.kernel,
    out_type=jax.ShapeDtypeStruct((NC, NP, CP), jnp.float32),
    mesh=_MESH,
    compiler_params=_SC_PARAMS,
    scratch_types=[
        pltpu.VMEM((NCHUNK, CHUNK), jnp.int32),
        pltpu.VMEM((CHUNK, CP), jnp.float32),
        pltpu.VMEM_SHARED((NP, CP), jnp.float32),
        pltpu.SemaphoreType.DMA,
    ],
)
def _deg_pass(dst_hbm, ones_hbm, zeros_hbm, out_hbm,
              dst_v, ones_v, acc_sh, dsem):
  cid = lax.axis_index("c")
  sid = lax.axis_index("s")
  wid = cid * NS + sid
  stripe = pl.ds(sid * RPT, RPT)
  pltpu.sync_copy(zeros_hbm.at[stripe], acc_sh.at[stripe])
  pltpu.sync_copy(dst_hbm.at[wid], dst_v)
  pltpu.sync_copy(ones_hbm, ones_v)
  plsc.subcore_barrier()

  def group(i, _):
    sd = {}
    for k in range(GRP):
      if k >= 3:
        sd[k - 3].wait()
      sd[k] = pltpu.async_copy(ones_v, acc_sh.at[dst_v.at[i * GRP + k]],
                               dsem, add=True)
    for k in range(GRP - 3, GRP):
      sd[k].wait()
    return 0

  lax.fori_loop(0, NCHUNK // GRP, group, 0)
  plsc.subcore_barrier()
  pltpu.sync_copy(acc_sh.at[stripe], out_hbm.at[cid, stripe])


def _dis_k(dq_ref, k):
  deg = 1.0 + dq_ref[0, :, CP * k:CP * k + 1] + dq_ref[1, :, CP * k:CP * k + 1]
  return lax.rsqrt(deg)


def _tc_h_body(x_ref, w1_ref, h_ref):
  h_ref[0:N, :] = jnp.dot(x_ref[...], w1_ref[...],
                          preferred_element_type=jnp.float32)


def _tc_g1_body(hq_ref, dq_ref, g1q_ref):
  disq = jnp.concatenate(
      [jnp.broadcast_to(_dis_k(dq_ref, k), (NPQ, HID)) for k in range(PK)],
      axis=1)
  g1q_ref[...] = hq_ref[...] * disq


def _tc2_body(ppv_ref, g1q_ref, dq_ref, b1_ref, w2q_ref, g2q_ref):
  h1 = []
  for k in range(PK):
    psum = jnp.concatenate(
        [ppv_ref[0, 0, :, HH * k:HH * (k + 1)]
         + ppv_ref[1, 0, :, HH * k:HH * (k + 1)],
         ppv_ref[0, 1, :, HH * k:HH * (k + 1)]
         + ppv_ref[1, 1, :, HH * k:HH * (k + 1)]], axis=1)
    g1k = g1q_ref[:, HID * k:HID * (k + 1)]
    agg = _dis_k(dq_ref, k) * (psum + g1k) + b1_ref[...]
    h1.append(jnp.maximum(agg, 0.0))
  h1q = jnp.concatenate(h1, axis=1)
  h2q = jnp.dot(h1q, w2q_ref[...], preferred_element_type=jnp.float32)
  disrow = jnp.concatenate(
      [jnp.broadcast_to(_dis_k(dq_ref, k), (NPQ, CP)) for k in range(PK)],
      axis=1)
  g2q_ref[...] = h2q * disrow


def _tc3_body(qqv_ref, g2q_ref, dq_ref, b2_ref, outq_ref):
  disrow = jnp.concatenate(
      [jnp.broadcast_to(_dis_k(dq_ref, k), (NPQ, CP)) for k in range(PK)],
      axis=1)
  outq_ref[...] = disrow * (qqv_ref[0] + qqv_ref[1] + g2q_ref[...]) \
      + b2_ref[...]


def _one_block(body, out_shape):
  return pl.pallas_call(
      body,
      out_shape=jax.ShapeDtypeStruct(out_shape, jnp.float32),
  )


def kernel(x, edge_index, W1, b1, W2, b2):
  src = edge_index[0]
  srca = (2 * src).reshape(NW, NCHUNK, CHUNK)
  srcb = (2 * src + 1).reshape(NW, NCHUNK, CHUNK)
  srcp = src.reshape(NW, NCHUNK, CHUNK)
  dst = edge_index[1].reshape(NW, NCHUNK, CHUNK)
  zeros64 = jnp.zeros((NP, HH), jnp.float32)
  zeros16 = jnp.zeros((NP, CP), jnp.float32)
  ones16 = jnp.ones((CHUNK, CP), jnp.float32)
  w2p = jnp.pad(W2, ((0, 0), (0, CP - C)))
  w2q = jnp.kron(jnp.eye(PK, dtype=jnp.float32), w2p)
  b1r = b1.reshape(1, HID)
  b2q = jnp.tile(jnp.pad(b2, (0, CP - C)), PK).reshape(1, PK * CP)

  degp = _deg_pass(dst, ones16, zeros16)
  degq = degp.reshape(NC, NPQ, PK * CP)

  h = _one_block(_tc_h_body, (NP, HID))(x, W1)
  hq = h.reshape(NPQ, PK * HID)
  g1q = _one_block(_tc_g1_body, (NPQ, PK * HID))(hq, degq)

  g2n = g1q.reshape(2 * NP, HH)
  pp1 = _edge_pass_wide(g2n, srca, srcb, dst, zeros64)
  ppv = pp1.reshape(NC, 2, NPQ, PK * HH)

  g2q = _one_block(_tc2_body, (NPQ, PK * CP))(ppv, g1q, degq, b1r, w2q)

  g2view = g2q.reshape(NP, CP)
  pp2 = _edge_pass_narrow(g2view, srcp, dst, zeros16)
  qqv = pp2.reshape(NC, NPQ, PK * CP)

  outq = _one_block(_tc3_body, (NPQ, PK * CP))(qqv, g2q, degq, b2q)
  return outq.reshape(NP, CP)[:N, :C]
